# baseline (speedup 1.0000x reference)
import functools

import jax
import jax.numpy as jnp
from jax import lax
from jax.experimental import pallas as pl
from jax.experimental.pallas import tpu as pltpu
from jax.experimental.pallas import tpu_sc as plsc

_NB = 4
_NO = 2


def _make_gather(V, D, B, NC, NS):
    NW = NC * NS
    C = 128
    L = 16
    G = C // L
    b_per_w = B // NW
    K = b_per_w // C
    TH = (V // C) * C
    PTH = TH // 2
    NAUX = (V - TH) // 2
    assert b_per_w * NW == B and K * C == b_per_w and K % _NB == 0

    mesh = plsc.VectorSubcoreMesh(core_axis_name="c", subcore_axis_name="s")

    @functools.partial(
        pl.kernel,
        mesh=mesh,
        compiler_params=pltpu.CompilerParams(needs_layout_passes=False),
        out_type=jax.ShapeDtypeStruct((D, B), jnp.float32),
        scratch_types=[
            pltpu.VMEM((K, C), jnp.int32),
            pltpu.VMEM((NAUX, 2 * D), jnp.float32),
            [pltpu.VMEM((C,), jnp.int32) for _ in range(_NB)],
            [pltpu.VMEM((C, 2 * D), jnp.float32) for _ in range(_NB)],
            [pltpu.VMEM((D, C), jnp.float32) for _ in range(_NO)],
            [pltpu.SemaphoreType.DMA for _ in range(_NB)],
            [pltpu.SemaphoreType.DMA for _ in range(_NO)],
        ],
    )
    def gather_kernel(r2d_hbm, idx_hbm, aux_hbm, outT_hbm,
                      idx_v, aux_v, pbufs, bufs, obufs, gsems, wsems):
        wid = lax.axis_index("s") * NC + lax.axis_index("c")
        base = wid * b_per_w
        pltpu.sync_copy(idx_hbm.at[wid], idx_v)
        pltpu.sync_copy(aux_hbm, aux_v)

        def fire_gather(g, b):
            for gi in range(G):
                pbufs[b][pl.ds(gi * L, L)] = (
                    lax.shift_right_logical(idx_v[g, pl.ds(gi * L, L)], 1))
            pltpu.make_async_copy(
                r2d_hbm.at[pbufs[b]], bufs[b], gsems[b]).start()

        def gather_wait(g, b):
            pltpu.make_async_copy(
                r2d_hbm.at[pbufs[b]], bufs[b], gsems[b]).wait()

        def writeback(g, o):
            return pltpu.make_async_copy(
                obufs[o], outT_hbm.at[:, pl.ds(base + g * C, C)], wsems[o])

        rows = [lax.iota(jnp.int32, L) + gi * L for gi in range(G)]

        for b in range(_NB - 1):
            fire_gather(b, b)

        @pl.loop(0, K, step=_NB)
        def _lap(j):
            for b in range(_NB):
                g = j + b
                o = b % _NO
                bp = (b - 1) % _NB

                if b == 0:
                    @pl.when(j >= 1)
                    def _wbwait0():
                        writeback(g - 1, (_NO - 1)).wait()
                else:
                    writeback(g - 1, (b - 1) % _NO).wait()
                @pl.when(g + _NB - 1 < K)
                def _refill():
                    fire_gather(g + _NB - 1, bp)

                gather_wait(g, b)
                cols = [(idx_v[g, pl.ds(gi * L, L)] & 1) * D for gi in range(G)]

                @pl.loop(0, L)
                def _diag(d):
                    cvec = (lax.iota(jnp.int32, L) + d) & (L - 1)
                    fvecs = [cvec + (k * L) for k in range(D // L)]
                    vs = [plsc.load_gather(
                              bufs[b], [rows[gi], cols[gi] + fvecs[k]])
                          for gi in range(G) for k in range(D // L)]
                    i = 0
                    for gi in range(G):
                        for k in range(D // L):
                            plsc.store_scatter(
                                obufs[o], [fvecs[k], rows[gi]], vs[i])
                            i += 1

                pvs = [lax.shift_right_logical(
                           idx_v[g, pl.ds(gi * L, L)], 1) for gi in range(G)]
                mx = pvs[0]
                for gi in range(1, G):
                    mx = jnp.maximum(mx, pvs[gi])
                @pl.when(jnp.max(mx) >= PTH)
                def _aux_fixup():
                    @pl.loop(0, G)
                    def _gi(gi):
                        ivec = idx_v[g, pl.ds(gi * L, L)]
                        pv = lax.shift_right_logical(ivec, 1)
                        tm = pv >= PTH
                        arow = jnp.maximum(pv - PTH, 0)
                        colb = (ivec & 1) * D
                        rowv = lax.iota(jnp.int32, L) + gi * L
                        @pl.loop(0, L)
                        def _d(d):
                            cvec = (lax.iota(jnp.int32, L) + d) & (L - 1)
                            for k in range(D // L):
                                fv = cvec + (k * L)
                                vm = plsc.load_gather(
                                    bufs[b], [rowv, colb + fv])
                                va = plsc.load_gather(
                                    aux_v, [arow, colb + fv])
                                v = jnp.where(tm, va, vm)
                                plsc.store_scatter(
                                    obufs[o], [fv, rowv], v)

                writeback(g, o).start()

        writeback(K - 1, (K - 1) % _NO).wait()

    return gather_kernel


def _make_table_transpose(V, D, NC, NS):
    NW = NC * NS
    L = 16
    CB = 128
    NBLK = V // CB
    P = V // 2
    M = D // L

    mesh = plsc.VectorSubcoreMesh(core_axis_name="c", subcore_axis_name="s")

    @functools.partial(
        pl.kernel,
        mesh=mesh,
        compiler_params=pltpu.CompilerParams(needs_layout_passes=False),
        out_type=jax.ShapeDtypeStruct((P, 2 * D), jnp.float32),
        scratch_types=[
            [pltpu.VMEM((D, CB), jnp.float32) for _ in range(2)],
            [pltpu.VMEM((CB // 2, 2 * D), jnp.float32) for _ in range(2)],
            [pltpu.SemaphoreType.DMA for _ in range(2)],
            [pltpu.SemaphoreType.DMA for _ in range(2)],
        ],
    )
    def transpose_kernel(tt_hbm, r2d_hbm, tbufs, obufs, gsems, wsems):
        wid = lax.axis_index("s") * NC + lax.axis_index("c")
        iota = lax.iota(jnp.int32, L)
        zeros = iota * 0

        def dma_in(rb, p):
            pltpu.make_async_copy(
                tt_hbm.at[:, pl.ds(rb * CB, CB)], tbufs[p], gsems[p]
            ).start()

        def dma_in_wait(rb, p):
            pltpu.make_async_copy(
                tt_hbm.at[:, pl.ds(rb * CB, CB)], tbufs[p], gsems[p]
            ).wait()

        def wb_full(rb, p):
            return pltpu.make_async_copy(
                obufs[p], r2d_hbm.at[pl.ds(rb * (CB // 2), CB // 2)],
                wsems[p])

        dma_in(wid, 0)

        @pl.loop(0, 246, step=2)
        def _lap(j):
            for s in range(2):
                p = s
                rb = wid + (j + s) * 32
                @pl.when(rb < NBLK)
                def _block():
                    @pl.when(rb + 32 < NBLK)
                    def _fire_next():
                        dma_in(rb + 32, 1 - p)
                    dma_in_wait(rb, p)
                    @pl.when(rb >= 64)
                    def _owait():
                        wb_full(rb - 64, p).wait()

                    @pl.loop(0, CB, unroll=4)
                    def _x(x):
                        xs = zeros + x
                        qs = zeros + lax.shift_right_logical(x, 1)
                        hoff = (x & 1) * D
                        cvecs = [((iota + x) & (L - 1)) + m * L
                                 for m in range(M)]
                        vals = [plsc.load_gather(tbufs[p], [cvecs[m], xs])
                                for m in range(M)]
                        for m in range(M):
                            plsc.store_scatter(
                                obufs[p], [qs, cvecs[m] + hoff], vals[m])

                    wb_full(rb, p).start()

        n = (NBLK - wid + 31) // 32
        for p in range(2):
            ip = ((n - 1 - p) // 2) * 2 + p
            rbp = wid + ip * 32
            @pl.when(n >= p + 1)
            def _drain():
                wb_full(rbp, p).wait()

    return transpose_kernel


def kernel(input, dim, index):
    table = input
    V, D = table.shape
    (B,) = index.shape
    info = plsc.get_sparse_core_info()
    NC, NS = info.num_cores, info.num_subcores
    NW = NC * NS
    C = 128
    idx3 = index.astype(jnp.int32).reshape(NW, (B // NW) // C, C)
    TH = (V // C) * C
    aux = table[TH:].reshape((V - TH) // 2, 2 * D)
    r2d = _make_table_transpose(V, D, NC, NS)(table.T)
    outT = _make_gather(V, D, B, NC, NS)(r2d, idx3, aux)
    return outT.T


# speedup vs baseline: 1.2118x; 1.2118x over previous
import functools

import jax
import jax.numpy as jnp
from jax import lax
from jax.experimental import pallas as pl
from jax.experimental.pallas import tpu as pltpu
from jax.experimental.pallas import tpu_sc as plsc

_NB = 4
_NO = 2


def _make_gather(V, D, B, NC, NS):
    NW = NC * NS
    C = 128
    L = 16
    G = C // L
    b_per_w = B // NW
    K = b_per_w // C
    TH = (V // C) * C
    PTH = TH // 2
    NAUX = (V - TH) // 2
    assert b_per_w * NW == B and K * C == b_per_w and K % _NB == 0

    mesh = plsc.VectorSubcoreMesh(core_axis_name="c", subcore_axis_name="s")

    @functools.partial(
        pl.kernel,
        mesh=mesh,
        compiler_params=pltpu.CompilerParams(needs_layout_passes=False),
        out_type=jax.ShapeDtypeStruct((D, B), jnp.float32),
        scratch_types=[
            pltpu.VMEM((K, C), jnp.int32),
            pltpu.VMEM((NAUX, 2 * D), jnp.float32),
            [pltpu.VMEM((C,), jnp.int32) for _ in range(_NB)],
            [pltpu.VMEM((C, 2 * D), jnp.float32) for _ in range(_NB)],
            [pltpu.VMEM((D, C), jnp.float32) for _ in range(_NO)],
            [pltpu.SemaphoreType.DMA for _ in range(_NB)],
            [pltpu.SemaphoreType.DMA for _ in range(_NO)],
        ],
    )
    def gather_kernel(r2d_hbm, idx_hbm, aux_hbm, outT_hbm,
                      idx_v, aux_v, pbufs, bufs, obufs, gsems, wsems):
        wid = lax.axis_index("s") * NC + lax.axis_index("c")
        base = wid * b_per_w
        pltpu.sync_copy(idx_hbm.at[wid], idx_v)
        pltpu.sync_copy(aux_hbm, aux_v)

        def fire_gather(g, b):
            for gi in range(G):
                pbufs[b][pl.ds(gi * L, L)] = (
                    lax.shift_right_logical(idx_v[g, pl.ds(gi * L, L)], 1))
            pltpu.make_async_copy(
                r2d_hbm.at[pbufs[b]], bufs[b], gsems[b]).start()

        def gather_wait(g, b):
            pltpu.make_async_copy(
                r2d_hbm.at[pbufs[b]], bufs[b], gsems[b]).wait()

        def writeback(g, o):
            return pltpu.make_async_copy(
                obufs[o], outT_hbm.at[:, pl.ds(base + g * C, C)], wsems[o])

        rows = [lax.iota(jnp.int32, L) + gi * L for gi in range(G)]

        for b in range(_NB - 1):
            fire_gather(b, b)

        @pl.loop(0, K, step=_NB)
        def _lap(j):
            for b in range(_NB):
                g = j + b
                o = b % _NO
                bp = (b - 1) % _NB

                if b == 0:
                    @pl.when(j >= 1)
                    def _wbwait0():
                        writeback(g - 1, (_NO - 1)).wait()
                else:
                    writeback(g - 1, (b - 1) % _NO).wait()
                @pl.when(g + _NB - 1 < K)
                def _refill():
                    fire_gather(g + _NB - 1, bp)

                gather_wait(g, b)
                cols = [(idx_v[g, pl.ds(gi * L, L)] & 1) * D for gi in range(G)]

                @pl.loop(0, L)
                def _diag(d):
                    cvec = (lax.iota(jnp.int32, L) + d) & (L - 1)
                    fvecs = [cvec + (k * L) for k in range(D // L)]
                    vs = [plsc.load_gather(
                              bufs[b], [rows[gi], cols[gi] + fvecs[k]])
                          for gi in range(G) for k in range(D // L)]
                    i = 0
                    for gi in range(G):
                        for k in range(D // L):
                            plsc.store_scatter(
                                obufs[o], [fvecs[k], rows[gi]], vs[i])
                            i += 1

                pvs = [lax.shift_right_logical(
                           idx_v[g, pl.ds(gi * L, L)], 1) for gi in range(G)]
                mx = pvs[0]
                for gi in range(1, G):
                    mx = jnp.maximum(mx, pvs[gi])
                @pl.when(jnp.max(mx) >= PTH)
                def _aux_fixup():
                    @pl.loop(0, G)
                    def _gi(gi):
                        ivec = idx_v[g, pl.ds(gi * L, L)]
                        pv = lax.shift_right_logical(ivec, 1)
                        tm = pv >= PTH
                        arow = jnp.maximum(pv - PTH, 0)
                        colb = (ivec & 1) * D
                        rowv = lax.iota(jnp.int32, L) + gi * L
                        @pl.loop(0, L)
                        def _d(d):
                            cvec = (lax.iota(jnp.int32, L) + d) & (L - 1)
                            for k in range(D // L):
                                fv = cvec + (k * L)
                                vm = plsc.load_gather(
                                    bufs[b], [rowv, colb + fv])
                                va = plsc.load_gather(
                                    aux_v, [arow, colb + fv])
                                v = jnp.where(tm, va, vm)
                                plsc.store_scatter(
                                    obufs[o], [fv, rowv], v)

                writeback(g, o).start()

        writeback(K - 1, (K - 1) % _NO).wait()

    return gather_kernel


def _make_table_transpose(V, D, NC, NS):
    NW = NC * NS
    L = 16
    CB = 128
    NBLK = V // CB
    P = V // 2
    M = D // L

    mesh = plsc.VectorSubcoreMesh(core_axis_name="c", subcore_axis_name="s")

    @functools.partial(
        pl.kernel,
        mesh=mesh,
        compiler_params=pltpu.CompilerParams(needs_layout_passes=False),
        out_type=jax.ShapeDtypeStruct((P, 2 * D), jnp.float32),
        scratch_types=[
            [pltpu.VMEM((D, CB), jnp.float32) for _ in range(2)],
            [pltpu.VMEM((CB // 2, 2 * D), jnp.float32) for _ in range(2)],
            [pltpu.SemaphoreType.DMA for _ in range(2)],
            [pltpu.SemaphoreType.DMA for _ in range(2)],
        ],
    )
    def transpose_kernel(tt_hbm, r2d_hbm, tbufs, obufs, gsems, wsems):
        wid = lax.axis_index("s") * NC + lax.axis_index("c")
        iota = lax.iota(jnp.int32, L)
        zeros = iota * 0

        def dma_in(rb, p):
            pltpu.make_async_copy(
                tt_hbm.at[:, pl.ds(rb * CB, CB)], tbufs[p], gsems[p]
            ).start()

        def dma_in_wait(rb, p):
            pltpu.make_async_copy(
                tt_hbm.at[:, pl.ds(rb * CB, CB)], tbufs[p], gsems[p]
            ).wait()

        def wb_full(rb, p):
            return pltpu.make_async_copy(
                obufs[p], r2d_hbm.at[pl.ds(rb * (CB // 2), CB // 2)],
                wsems[p])

        dma_in(wid, 0)

        @pl.loop(0, 246, step=2)
        def _lap(j):
            for s in range(2):
                p = s
                rb = wid + (j + s) * 32
                @pl.when(rb < NBLK)
                def _block():
                    @pl.when(rb + 32 < NBLK)
                    def _fire_next():
                        dma_in(rb + 32, 1 - p)
                    dma_in_wait(rb, p)
                    @pl.when(rb >= 64)
                    def _owait():
                        wb_full(rb - 64, p).wait()

                    @pl.loop(0, CB, step=8)
                    def _x8(x0):
                        per_x = []
                        for xi in range(8):
                            x = x0 + xi
                            xs = zeros + x
                            qs = zeros + lax.shift_right_logical(x, 1)
                            hoff = (x & 1) * D
                            cvecs = [((iota + x) & (L - 1)) + m * L
                                     for m in range(M)]
                            per_x.append((qs, hoff, cvecs))
                            for m in range(M):
                                per_x.append(
                                    plsc.load_gather(
                                        tbufs[p], [cvecs[m], xs]))
                        i = 0
                        for xi in range(8):
                            qs, hoff, cvecs = per_x[i]; i += 1
                            for m in range(M):
                                plsc.store_scatter(
                                    obufs[p], [qs, cvecs[m] + hoff],
                                    per_x[i])
                                i += 1

                    wb_full(rb, p).start()

        n = (NBLK - wid + 31) // 32
        for p in range(2):
            ip = ((n - 1 - p) // 2) * 2 + p
            rbp = wid + ip * 32
            @pl.when(n >= p + 1)
            def _drain():
                wb_full(rbp, p).wait()

    return transpose_kernel


def kernel(input, dim, index):
    table = input
    V, D = table.shape
    (B,) = index.shape
    info = plsc.get_sparse_core_info()
    NC, NS = info.num_cores, info.num_subcores
    NW = NC * NS
    C = 128
    idx3 = index.astype(jnp.int32).reshape(NW, (B // NW) // C, C)
    TH = (V // C) * C
    aux = table[TH:].reshape((V - TH) // 2, 2 * D)
    r2d = _make_table_transpose(V, D, NC, NS)(table.T)
    outT = _make_gather(V, D, B, NC, NS)(r2d, idx3, aux)
    return outT.T


# speedup vs baseline: 2.7727x; 2.2881x over previous
import functools

import jax
import jax.numpy as jnp
from jax import lax
from jax.experimental import pallas as pl
from jax.experimental.pallas import tpu as pltpu
from jax.experimental.pallas import tpu_sc as plsc

_NB = 4
_NO = 2


def _make_gather(V, D, B, NC, NS):
    NW = NC * NS
    C = 128
    L = 16
    G = C // L
    b_per_w = B // NW
    K = b_per_w // C
    TH = (V // C) * C
    PTH = TH // 2
    NAUX = (V - TH) // 2
    assert b_per_w * NW == B and K * C == b_per_w and K % _NB == 0

    mesh = plsc.VectorSubcoreMesh(core_axis_name="c", subcore_axis_name="s")

    @functools.partial(
        pl.kernel,
        mesh=mesh,
        compiler_params=pltpu.CompilerParams(needs_layout_passes=False),
        out_type=jax.ShapeDtypeStruct((D, B), jnp.float32),
        scratch_types=[
            pltpu.VMEM((K, C), jnp.int32),
            pltpu.VMEM((NAUX, 2 * D), jnp.float32),
            [pltpu.VMEM((C,), jnp.int32) for _ in range(_NB)],
            [pltpu.VMEM((C, 2 * D), jnp.float32) for _ in range(_NB)],
            [pltpu.VMEM((D, C), jnp.float32) for _ in range(_NO)],
            [pltpu.SemaphoreType.DMA for _ in range(_NB)],
            [pltpu.SemaphoreType.DMA for _ in range(_NO)],
        ],
    )
    def gather_kernel(r2d_hbm, idx_hbm, aux_hbm, outT_hbm,
                      idx_v, aux_v, pbufs, bufs, obufs, gsems, wsems):
        wid = lax.axis_index("s") * NC + lax.axis_index("c")
        base = wid * b_per_w
        pltpu.sync_copy(idx_hbm.at[wid], idx_v)
        pltpu.sync_copy(aux_hbm, aux_v)

        def fire_gather(g, b):
            for gi in range(G):
                pbufs[b][pl.ds(gi * L, L)] = (
                    lax.shift_right_logical(idx_v[g, pl.ds(gi * L, L)], 1))
            pltpu.make_async_copy(
                r2d_hbm.at[pbufs[b]], bufs[b], gsems[b]).start()

        def gather_wait(g, b):
            pltpu.make_async_copy(
                r2d_hbm.at[pbufs[b]], bufs[b], gsems[b]).wait()

        def writeback(g, o):
            return pltpu.make_async_copy(
                obufs[o], outT_hbm.at[:, pl.ds(base + g * C, C)], wsems[o])

        rows = [lax.iota(jnp.int32, L) + gi * L for gi in range(G)]

        for b in range(_NB - 1):
            fire_gather(b, b)

        @pl.loop(0, K, step=_NB)
        def _lap(j):
            for b in range(_NB):
                g = j + b
                o = b % _NO
                bp = (b - 1) % _NB

                if b == 0:
                    @pl.when(j >= 1)
                    def _wbwait0():
                        writeback(g - 1, (_NO - 1)).wait()
                else:
                    writeback(g - 1, (b - 1) % _NO).wait()
                @pl.when(g + _NB - 1 < K)
                def _refill():
                    fire_gather(g + _NB - 1, bp)

                gather_wait(g, b)
                cols = [(idx_v[g, pl.ds(gi * L, L)] & 1) * D for gi in range(G)]

                @pl.loop(0, L)
                def _diag(d):
                    cvec = (lax.iota(jnp.int32, L) + d) & (L - 1)
                    fvecs = [cvec + (k * L) for k in range(D // L)]
                    vs = [plsc.load_gather(
                              bufs[b], [rows[gi], cols[gi] + fvecs[k]])
                          for gi in range(G) for k in range(D // L)]
                    i = 0
                    for gi in range(G):
                        for k in range(D // L):
                            plsc.store_scatter(
                                obufs[o], [fvecs[k], rows[gi]], vs[i])
                            i += 1

                pvs = [lax.shift_right_logical(
                           idx_v[g, pl.ds(gi * L, L)], 1) for gi in range(G)]
                mx = pvs[0]
                for gi in range(1, G):
                    mx = jnp.maximum(mx, pvs[gi])
                @pl.when(jnp.max(mx) >= PTH)
                def _aux_fixup():
                    @pl.loop(0, G)
                    def _gi(gi):
                        ivec = idx_v[g, pl.ds(gi * L, L)]
                        pv = lax.shift_right_logical(ivec, 1)
                        tm = pv >= PTH
                        arow = jnp.maximum(pv - PTH, 0)
                        colb = (ivec & 1) * D
                        rowv = lax.iota(jnp.int32, L) + gi * L
                        @pl.loop(0, L)
                        def _d(d):
                            cvec = (lax.iota(jnp.int32, L) + d) & (L - 1)
                            for k in range(D // L):
                                fv = cvec + (k * L)
                                vm = plsc.load_gather(
                                    bufs[b], [rowv, colb + fv])
                                va = plsc.load_gather(
                                    aux_v, [arow, colb + fv])
                                v = jnp.where(tm, va, vm)
                                plsc.store_scatter(
                                    obufs[o], [fv, rowv], v)

                writeback(g, o).start()

        writeback(K - 1, (K - 1) % _NO).wait()

    return gather_kernel


def _make_table_transpose(V, D, NC, NS):
    NW = NC * NS
    L = 16
    CB = 128
    NBLK = V // CB
    P = V // 2
    M = D // L

    mesh = plsc.VectorSubcoreMesh(core_axis_name="c", subcore_axis_name="s")

    @functools.partial(
        pl.kernel,
        mesh=mesh,
        compiler_params=pltpu.CompilerParams(needs_layout_passes=False),
        out_type=jax.ShapeDtypeStruct((P, 2 * D), jnp.float32),
        scratch_types=[
            [pltpu.VMEM((D, CB), jnp.float32) for _ in range(2)],
            [pltpu.VMEM((CB // 2, 2 * D), jnp.float32) for _ in range(2)],
            [pltpu.SemaphoreType.DMA for _ in range(2)],
            [pltpu.SemaphoreType.DMA for _ in range(2)],
        ],
    )
    def transpose_kernel(tt_hbm, r2d_hbm, tbufs, obufs, gsems, wsems):
        wid = lax.axis_index("s") * NC + lax.axis_index("c")
        iota = lax.iota(jnp.int32, L)
        zeros = iota * 0

        def dma_in(rb, p):
            pltpu.make_async_copy(
                tt_hbm.at[:, pl.ds(rb * CB, CB)], tbufs[p], gsems[p]
            ).start()

        def dma_in_wait(rb, p):
            pltpu.make_async_copy(
                tt_hbm.at[:, pl.ds(rb * CB, CB)], tbufs[p], gsems[p]
            ).wait()

        def wb_full(rb, p):
            return pltpu.make_async_copy(
                obufs[p], r2d_hbm.at[pl.ds(rb * (CB // 2), CB // 2)],
                wsems[p])

        dma_in(wid, 0)

        @pl.loop(0, 246, step=2)
        def _lap(j):
            for s in range(2):
                p = s
                rb = wid + (j + s) * 32
                @pl.when(rb < NBLK)
                def _block():
                    @pl.when(rb + 32 < NBLK)
                    def _fire_next():
                        dma_in(rb + 32, 1 - p)
                    dma_in_wait(rb, p)
                    @pl.when(rb >= 64)
                    def _owait():
                        wb_full(rb - 64, p).wait()

                    @pl.loop(0, CB, step=L)
                    def _xg(x0):
                        xvec = iota + x0
                        qvec = lax.shift_right_logical(xvec, 1)
                        hoff = (xvec & 1) * D
                        for m in range(M):
                            cvs = [((iota + d) & (L - 1)) + m * L
                                   for d in range(L)]
                            vals = [plsc.load_gather(
                                        tbufs[p], [cvs[d], xvec])
                                    for d in range(L)]
                            for d in range(L):
                                plsc.store_scatter(
                                    obufs[p], [qvec, hoff + cvs[d]],
                                    vals[d])

                    wb_full(rb, p).start()

        n = (NBLK - wid + 31) // 32
        for p in range(2):
            ip = ((n - 1 - p) // 2) * 2 + p
            rbp = wid + ip * 32
            @pl.when(n >= p + 1)
            def _drain():
                wb_full(rbp, p).wait()

    return transpose_kernel


def kernel(input, dim, index):
    table = input
    V, D = table.shape
    (B,) = index.shape
    info = plsc.get_sparse_core_info()
    NC, NS = info.num_cores, info.num_subcores
    NW = NC * NS
    C = 128
    idx3 = index.astype(jnp.int32).reshape(NW, (B // NW) // C, C)
    TH = (V // C) * C
    aux = table[TH:].reshape((V - TH) // 2, 2 * D)
    r2d = _make_table_transpose(V, D, NC, NS)(table.T)
    outT = _make_gather(V, D, B, NC, NS)(r2d, idx3, aux)
    return outT.T


# speedup vs baseline: 3.0754x; 1.1092x over previous
import functools

import jax
import jax.numpy as jnp
from jax import lax
from jax.experimental import pallas as pl
from jax.experimental.pallas import tpu as pltpu
from jax.experimental.pallas import tpu_sc as plsc

_NB = 4
_NO = 2


def _make_gather(V, D, B, NC, NS):
    NW = NC * NS
    C = 128
    L = 16
    G = C // L
    b_per_w = B // NW
    K = b_per_w // C
    TH = (V // C) * C
    PTH = TH // 2
    NAUX = (V - TH) // 2
    assert b_per_w * NW == B and K * C == b_per_w and K % _NB == 0

    mesh = plsc.VectorSubcoreMesh(core_axis_name="c", subcore_axis_name="s")

    @functools.partial(
        pl.kernel,
        mesh=mesh,
        compiler_params=pltpu.CompilerParams(needs_layout_passes=False),
        out_type=jax.ShapeDtypeStruct((D, B), jnp.float32),
        scratch_types=[
            pltpu.VMEM((K, C), jnp.int32),
            pltpu.VMEM((NAUX, 2 * D), jnp.float32),
            [pltpu.VMEM((C,), jnp.int32) for _ in range(_NB)],
            [pltpu.VMEM((C, 2 * D), jnp.float32) for _ in range(_NB)],
            [pltpu.VMEM((D, C), jnp.float32) for _ in range(_NO)],
            [pltpu.SemaphoreType.DMA for _ in range(_NB)],
            [pltpu.SemaphoreType.DMA for _ in range(_NO)],
        ],
    )
    def gather_kernel(r2d_hbm, idx_hbm, aux_hbm, outT_hbm,
                      idx_v, aux_v, pbufs, bufs, obufs, gsems, wsems):
        wid = lax.axis_index("s") * NC + lax.axis_index("c")
        base = wid * b_per_w
        pltpu.sync_copy(idx_hbm.at[wid], idx_v)
        pltpu.sync_copy(aux_hbm, aux_v)

        def fire_gather(g, b):
            for gi in range(G):
                pbufs[b][pl.ds(gi * L, L)] = (
                    lax.shift_right_logical(idx_v[g, pl.ds(gi * L, L)], 1))
            pltpu.make_async_copy(
                r2d_hbm.at[pbufs[b]], bufs[b], gsems[b]).start()

        def gather_wait(g, b):
            pltpu.make_async_copy(
                r2d_hbm.at[pbufs[b]], bufs[b], gsems[b]).wait()

        def writeback(g, o):
            return pltpu.make_async_copy(
                obufs[o], outT_hbm.at[:, pl.ds(base + g * C, C)], wsems[o])

        rows = [lax.iota(jnp.int32, L) + gi * L for gi in range(G)]

        for b in range(_NB - 1):
            fire_gather(b, b)

        @pl.loop(0, K, step=_NB)
        def _lap(j):
            for b in range(_NB):
                g = j + b
                o = b % _NO
                bp = (b - 1) % _NB

                if b == 0:
                    @pl.when(j >= 1)
                    def _wbwait0():
                        writeback(g - 1, (_NO - 1)).wait()
                else:
                    writeback(g - 1, (b - 1) % _NO).wait()
                @pl.when(g + _NB - 1 < K)
                def _refill():
                    fire_gather(g + _NB - 1, bp)

                gather_wait(g, b)
                cols = [(idx_v[g, pl.ds(gi * L, L)] & 1) * D for gi in range(G)]

                @pl.loop(0, L)
                def _diag(d):
                    cvec = (lax.iota(jnp.int32, L) + d) & (L - 1)
                    fvecs = [cvec + (k * L) for k in range(D // L)]
                    vs = [plsc.load_gather(
                              bufs[b], [rows[gi], cols[gi] + fvecs[k]])
                          for gi in range(G) for k in range(D // L)]
                    i = 0
                    for gi in range(G):
                        for k in range(D // L):
                            plsc.store_scatter(
                                obufs[o], [fvecs[k], rows[gi]], vs[i])
                            i += 1

                pvs = [lax.shift_right_logical(
                           idx_v[g, pl.ds(gi * L, L)], 1) for gi in range(G)]
                mx = pvs[0]
                for gi in range(1, G):
                    mx = jnp.maximum(mx, pvs[gi])
                @pl.when(jnp.max(mx) >= PTH)
                def _aux_fixup():
                    @pl.loop(0, G)
                    def _gi(gi):
                        ivec = idx_v[g, pl.ds(gi * L, L)]
                        pv = lax.shift_right_logical(ivec, 1)
                        tm = pv >= PTH
                        arow = jnp.maximum(pv - PTH, 0)
                        colb = (ivec & 1) * D
                        rowv = lax.iota(jnp.int32, L) + gi * L
                        @pl.loop(0, L)
                        def _d(d):
                            cvec = (lax.iota(jnp.int32, L) + d) & (L - 1)
                            for k in range(D // L):
                                fv = cvec + (k * L)
                                vm = plsc.load_gather(
                                    bufs[b], [rowv, colb + fv])
                                va = plsc.load_gather(
                                    aux_v, [arow, colb + fv])
                                v = jnp.where(tm, va, vm)
                                plsc.store_scatter(
                                    obufs[o], [fv, rowv], v)

                writeback(g, o).start()

        writeback(K - 1, (K - 1) % _NO).wait()

    return gather_kernel


def _make_table_transpose(V, D, NC, NS):
    NW = NC * NS
    L = 16
    CB = 256
    NBLK = V // CB
    P = V // 2
    M = D // L

    mesh = plsc.VectorSubcoreMesh(core_axis_name="c", subcore_axis_name="s")

    @functools.partial(
        pl.kernel,
        mesh=mesh,
        compiler_params=pltpu.CompilerParams(needs_layout_passes=False),
        out_type=jax.ShapeDtypeStruct((P, 2 * D), jnp.float32),
        scratch_types=[
            [pltpu.VMEM((D, CB), jnp.float32) for _ in range(2)],
            [pltpu.VMEM((CB // 2, 2 * D), jnp.float32) for _ in range(2)],
            [pltpu.SemaphoreType.DMA for _ in range(2)],
            [pltpu.SemaphoreType.DMA for _ in range(2)],
        ],
    )
    def transpose_kernel(tt_hbm, r2d_hbm, tbufs, obufs, gsems, wsems):
        wid = lax.axis_index("s") * NC + lax.axis_index("c")
        iota = lax.iota(jnp.int32, L)
        zeros = iota * 0

        def dma_in(rb, p):
            pltpu.make_async_copy(
                tt_hbm.at[:, pl.ds(rb * CB, CB)], tbufs[p], gsems[p]
            ).start()

        def dma_in_wait(rb, p):
            pltpu.make_async_copy(
                tt_hbm.at[:, pl.ds(rb * CB, CB)], tbufs[p], gsems[p]
            ).wait()

        def wb_full(rb, p):
            return pltpu.make_async_copy(
                obufs[p], r2d_hbm.at[pl.ds(rb * (CB // 2), CB // 2)],
                wsems[p])

        dma_in(wid, 0)

        NL = ((NBLK + 31) // 32 + 1) // 2 * 2
        @pl.loop(0, NL, step=2)
        def _lap(j):
            for s in range(2):
                p = s
                rb = wid + (j + s) * 32
                @pl.when(rb < NBLK)
                def _block():
                    @pl.when(rb + 32 < NBLK)
                    def _fire_next():
                        dma_in(rb + 32, 1 - p)
                    dma_in_wait(rb, p)
                    @pl.when(rb >= 64)
                    def _owait():
                        wb_full(rb - 64, p).wait()

                    @pl.loop(0, CB, step=L)
                    def _xg(x0):
                        xvec = iota + x0
                        qvec = lax.shift_right_logical(xvec, 1)
                        hoff = (xvec & 1) * D
                        for m in range(M):
                            cvs = [((iota + d) & (L - 1)) + m * L
                                   for d in range(L)]
                            vals = [plsc.load_gather(
                                        tbufs[p], [cvs[d], xvec])
                                    for d in range(L)]
                            for d in range(L):
                                plsc.store_scatter(
                                    obufs[p], [qvec, hoff + cvs[d]],
                                    vals[d])

                    wb_full(rb, p).start()

        n = (NBLK - wid + 31) // 32
        for p in range(2):
            ip = ((n - 1 - p) // 2) * 2 + p
            rbp = wid + ip * 32
            @pl.when(n >= p + 1)
            def _drain():
                wb_full(rbp, p).wait()

    return transpose_kernel


def kernel(input, dim, index):
    table = input
    V, D = table.shape
    (B,) = index.shape
    info = plsc.get_sparse_core_info()
    NC, NS = info.num_cores, info.num_subcores
    NW = NC * NS
    C = 128
    idx3 = index.astype(jnp.int32).reshape(NW, (B // NW) // C, C)
    TH = (V // C) * C
    aux = table[TH:].reshape((V - TH) // 2, 2 * D)
    r2d = _make_table_transpose(V, D, NC, NS)(table.T)
    outT = _make_gather(V, D, B, NC, NS)(r2d, idx3, aux)
    return outT.T
